# baseline (speedup 1.0000x reference)
import functools

import jax
import jax.numpy as jnp
from jax.experimental import pallas as pl
from jax.experimental.pallas import tpu as pltpu

N = 10000
NP = 10240


def _mm_kernel(h_ref, w_ref, o_ref):
    o_ref[...] = jnp.dot(h_ref[...], w_ref[...],
                         preferred_element_type=jnp.float32
                         ).astype(o_ref.dtype)


def _matmul(h, w, bm=2048, out_dtype=jnp.float32):
    m, k = h.shape
    n = w.shape[1]
    return pl.pallas_call(
        _mm_kernel,
        grid=(m // bm,),
        in_specs=[
            pl.BlockSpec((bm, k), lambda i: (i, 0)),
            pl.BlockSpec((k, n), lambda i: (0, 0)),
        ],
        out_specs=pl.BlockSpec((bm, n), lambda i: (i, 0)),
        out_shape=jax.ShapeDtypeStruct((m, n), out_dtype),
        compiler_params=pltpu.CompilerParams(
            dimension_semantics=("parallel",)),
    )(h, w)


def _agg_kernel(a_ref, hw_ref, b_ref, o_ref):
    k = pl.program_id(1)

    @pl.when(k == 0)
    def _init():
        o_ref[...] = jnp.zeros_like(o_ref)

    o_ref[...] += jnp.dot(a_ref[...], hw_ref[...],
                          preferred_element_type=jnp.float32)

    @pl.when(k == pl.num_programs(1) - 1)
    def _fin():
        o_ref[...] = jnp.maximum(o_ref[...] + b_ref[...], 0.0)


def _aggregate(a, hw, b, bm=1024, bk=2048):
    m = a.shape[0]
    n = hw.shape[1]
    return pl.pallas_call(
        _agg_kernel,
        grid=(m // bm, m // bk),
        in_specs=[
            pl.BlockSpec((bm, bk), lambda i, k: (i, k)),
            pl.BlockSpec((bk, n), lambda i, k: (k, 0)),
            pl.BlockSpec((1, n), lambda i, k: (0, 0)),
        ],
        out_specs=pl.BlockSpec((bm, n), lambda i, k: (i, 0)),
        out_shape=jax.ShapeDtypeStruct((m, n), jnp.float32),
        compiler_params=pltpu.CompilerParams(
            dimension_semantics=("parallel", "arbitrary")),
    )(a, hw, b)


def _outer_kernel(vr_ref, vt_ref, o_ref):
    o_ref[...] = jnp.maximum(vr_ref[...] * vt_ref[...], 0.0)


def _outer_relu(v, bm=400):
    vt = v.reshape(1, N)
    return pl.pallas_call(
        _outer_kernel,
        grid=(N // bm,),
        in_specs=[
            pl.BlockSpec((bm, 1), lambda i: (i, 0)),
            pl.BlockSpec((1, N), lambda i: (0, 0)),
        ],
        out_specs=pl.BlockSpec((bm, N), lambda i: (i, 0)),
        out_shape=jax.ShapeDtypeStruct((N, N), jnp.float32),
        compiler_params=pltpu.CompilerParams(
            dimension_semantics=("parallel",)),
    )(v, vt)


def kernel(x, edge_index, W1, b1, W2, b2, W3, b3, W4, b4, Wout, bout):
    src = edge_index[0].astype(jnp.int32)
    dst = edge_index[1].astype(jnp.int32)

    ones = jnp.ones(src.shape, jnp.float32)
    deg = jnp.ones((N,), jnp.float32).at[dst].add(ones)
    dinv = jax.lax.rsqrt(deg)
    norm = dinv[src] * dinv[dst]

    a = jnp.zeros((NP, NP), jnp.bfloat16)
    a = a.at[dst, src].add(norm.astype(jnp.bfloat16))
    diag = jnp.arange(N, dtype=jnp.int32)
    a = a.at[diag, diag].add((dinv * dinv).astype(jnp.bfloat16))

    h = jnp.pad(x, ((0, NP - N), (0, 0)))
    for w, b in ((W1, b1), (W2, b2), (W3, b3), (W4, b4)):
        hw = _matmul(h, w, out_dtype=jnp.bfloat16)
        h = _aggregate(a, hw, b.reshape(1, -1))

    v = _matmul(h, Wout)
    v = (v + bout)[:N]
    return _outer_relu(v)


# speedup vs baseline: 1.4938x; 1.4938x over previous
import functools

import jax
import jax.numpy as jnp
from jax.experimental import pallas as pl
from jax.experimental.pallas import tpu as pltpu

N = 10000
NP = 10240


def _mm_kernel(h_ref, w_ref, o_ref):
    o_ref[...] = jnp.dot(h_ref[...], w_ref[...],
                         preferred_element_type=jnp.float32
                         ).astype(o_ref.dtype)


def _matmul(h, w, bm=2048, out_dtype=jnp.float32):
    m, k = h.shape
    n = w.shape[1]
    return pl.pallas_call(
        _mm_kernel,
        grid=(m // bm,),
        in_specs=[
            pl.BlockSpec((bm, k), lambda i: (i, 0)),
            pl.BlockSpec((k, n), lambda i: (0, 0)),
        ],
        out_specs=pl.BlockSpec((bm, n), lambda i: (i, 0)),
        out_shape=jax.ShapeDtypeStruct((m, n), out_dtype),
        compiler_params=pltpu.CompilerParams(
            dimension_semantics=("parallel",)),
    )(h, w)


def _agg_kernel(a_ref, hw_ref, b_ref, o_ref):
    k = pl.program_id(1)

    @pl.when(k == 0)
    def _init():
        o_ref[...] = jnp.zeros_like(o_ref)

    o_ref[...] += jnp.dot(a_ref[...].astype(jnp.bfloat16), hw_ref[...],
                          preferred_element_type=jnp.float32)

    @pl.when(k == pl.num_programs(1) - 1)
    def _fin():
        o_ref[...] = jnp.maximum(o_ref[...] + b_ref[...], 0.0)


def _aggregate(a, hw, b, bm=1024, bk=2048):
    m = a.shape[0]
    n = hw.shape[1]
    return pl.pallas_call(
        _agg_kernel,
        grid=(m // bm, m // bk),
        in_specs=[
            pl.BlockSpec((bm, bk), lambda i, k: (i, k)),
            pl.BlockSpec((bk, n), lambda i, k: (k, 0)),
            pl.BlockSpec((1, n), lambda i, k: (0, 0)),
        ],
        out_specs=pl.BlockSpec((bm, n), lambda i, k: (i, 0)),
        out_shape=jax.ShapeDtypeStruct((m, n), jnp.float32),
        compiler_params=pltpu.CompilerParams(
            dimension_semantics=("parallel", "arbitrary")),
    )(a, hw, b)


def _outer_kernel(vr_ref, vt_ref, o_ref):
    o_ref[...] = jnp.maximum(vr_ref[...] * vt_ref[...], 0.0)


def _outer_relu(v, bm=400):
    vt = v.reshape(1, N)
    return pl.pallas_call(
        _outer_kernel,
        grid=(N // bm,),
        in_specs=[
            pl.BlockSpec((bm, 1), lambda i: (i, 0)),
            pl.BlockSpec((1, N), lambda i: (0, 0)),
        ],
        out_specs=pl.BlockSpec((bm, N), lambda i: (i, 0)),
        out_shape=jax.ShapeDtypeStruct((N, N), jnp.float32),
        compiler_params=pltpu.CompilerParams(
            dimension_semantics=("parallel",)),
    )(v, vt)


def kernel(x, edge_index, W1, b1, W2, b2, W3, b3, W4, b4, Wout, bout):
    src = edge_index[0].astype(jnp.int32)
    dst = edge_index[1].astype(jnp.int32)

    ones = jnp.ones(src.shape, jnp.float32)
    deg = jnp.ones((N,), jnp.float32).at[dst].add(ones)
    dinv = jax.lax.rsqrt(deg)
    norm = dinv[src] * dinv[dst]

    a = jnp.zeros((NP, NP), jnp.float32)
    a = a.at[dst, src].add(norm)
    diag = jnp.arange(N, dtype=jnp.int32)
    a = a.at[diag, diag].add(dinv * dinv)

    h = jnp.pad(x, ((0, NP - N), (0, 0)))
    for w, b in ((W1, b1), (W2, b2), (W3, b3), (W4, b4)):
        hw = _matmul(h, w, out_dtype=jnp.bfloat16)
        h = _aggregate(a, hw, b.reshape(1, -1))

    v = _matmul(h, Wout)
    v = (v + bout)[:N]
    return _outer_relu(v)


# speedup vs baseline: 2.9061x; 1.9454x over previous
import jax
import jax.numpy as jnp
from jax.experimental import pallas as pl
from jax.experimental.pallas import tpu as pltpu

N = 10000
NP = 10240


def _mm_kernel(h_ref, d_ref, w_ref, o_ref):
    o_ref[...] = jnp.dot(h_ref[...] * d_ref[...], w_ref[...],
                         preferred_element_type=jnp.float32
                         ).astype(o_ref.dtype)


def _scaled_matmul(h, dinv, w, bm=2048, out_dtype=jnp.bfloat16):
    m, k = h.shape
    n = w.shape[1]
    return pl.pallas_call(
        _mm_kernel,
        grid=(m // bm,),
        in_specs=[
            pl.BlockSpec((bm, k), lambda i: (i, 0)),
            pl.BlockSpec((bm, 1), lambda i: (i, 0)),
            pl.BlockSpec((k, n), lambda i: (0, 0)),
        ],
        out_specs=pl.BlockSpec((bm, n), lambda i: (i, 0)),
        out_shape=jax.ShapeDtypeStruct((m, n), out_dtype),
        compiler_params=pltpu.CompilerParams(
            dimension_semantics=("parallel",)),
    )(h, dinv, w)


def _agg_kernel(a_ref, hw_ref, d_ref, b_ref, o_ref):
    i = pl.program_id(0)
    k = pl.program_id(1)

    @pl.when(k == 0)
    def _init():
        o_ref[...] = jnp.zeros_like(o_ref)

    @pl.when(k == i)
    def _self_loop():
        o_ref[...] += hw_ref[...].astype(jnp.float32)

    o_ref[...] += jnp.dot(a_ref[...].astype(jnp.bfloat16), hw_ref[...],
                          preferred_element_type=jnp.float32)

    @pl.when(k == pl.num_programs(1) - 1)
    def _fin():
        o_ref[...] = jnp.maximum(d_ref[...] * o_ref[...] + b_ref[...], 0.0)


def _aggregate(a, hw, dinv, b, bm=1024):
    m = a.shape[0]
    n = hw.shape[1]
    return pl.pallas_call(
        _agg_kernel,
        grid=(m // bm, m // bm),
        in_specs=[
            pl.BlockSpec((bm, bm), lambda i, k: (i, k)),
            pl.BlockSpec((bm, n), lambda i, k: (k, 0)),
            pl.BlockSpec((bm, 1), lambda i, k: (i, 0)),
            pl.BlockSpec((1, n), lambda i, k: (0, 0)),
        ],
        out_specs=pl.BlockSpec((bm, n), lambda i, k: (i, 0)),
        out_shape=jax.ShapeDtypeStruct((m, n), jnp.float32),
        compiler_params=pltpu.CompilerParams(
            dimension_semantics=("parallel", "arbitrary")),
    )(a, hw, dinv, b)


def _proj_kernel(h_ref, w_ref, o_ref):
    o_ref[...] = jnp.dot(h_ref[...], w_ref[...],
                         preferred_element_type=jnp.float32)


def _outer_kernel(vr_ref, vt_ref, o_ref):
    o_ref[...] = jnp.maximum(vr_ref[...] * vt_ref[...], 0.0)


def _outer_relu(v, bm=400):
    vt = v.reshape(1, N)
    return pl.pallas_call(
        _outer_kernel,
        grid=(N // bm,),
        in_specs=[
            pl.BlockSpec((bm, 1), lambda i: (i, 0)),
            pl.BlockSpec((1, N), lambda i: (0, 0)),
        ],
        out_specs=pl.BlockSpec((bm, N), lambda i: (i, 0)),
        out_shape=jax.ShapeDtypeStruct((N, N), jnp.float32),
        compiler_params=pltpu.CompilerParams(
            dimension_semantics=("parallel",)),
    )(v, vt)


def kernel(x, edge_index, W1, b1, W2, b2, W3, b3, W4, b4, Wout, bout):
    src = edge_index[0].astype(jnp.int32)
    dst = edge_index[1].astype(jnp.int32)

    ones = jnp.ones(src.shape, jnp.float32)
    deg = jnp.ones((NP,), jnp.float32).at[dst].add(ones)
    dinv = jax.lax.rsqrt(deg).reshape(NP, 1)

    flat = jnp.zeros((NP * NP,), jnp.float32)
    flat = flat.at[dst * NP + src].add(ones)
    a = flat.reshape(NP, NP)

    h = jnp.pad(x, ((0, NP - N), (0, 0)))
    for w, b in ((W1, b1), (W2, b2), (W3, b3), (W4, b4)):
        hw = _scaled_matmul(h, dinv, w)
        h = _aggregate(a, hw, dinv, b.reshape(1, -1))

    v = pl.pallas_call(
        _proj_kernel,
        in_specs=[pl.BlockSpec((NP, h.shape[1]), lambda: (0, 0)),
                  pl.BlockSpec((h.shape[1], 1), lambda: (0, 0))],
        out_specs=pl.BlockSpec((NP, 1), lambda: (0, 0)),
        out_shape=jax.ShapeDtypeStruct((NP, 1), jnp.float32),
    )(h, Wout)
    v = (v + bout)[:N]
    return _outer_relu(v)


# speedup vs baseline: 3.1282x; 1.0764x over previous
import jax
import jax.numpy as jnp
from jax.experimental import pallas as pl
from jax.experimental.pallas import tpu as pltpu

N = 10000
NP = 10240


def _mm_kernel(h_ref, d_ref, w_ref, o_ref):
    o_ref[...] = jnp.dot(h_ref[...] * d_ref[...], w_ref[...],
                         preferred_element_type=jnp.float32
                         ).astype(o_ref.dtype)


def _scaled_matmul(h, dinv, w, bm=2048, out_dtype=jnp.bfloat16):
    m, k = h.shape
    n = w.shape[1]
    return pl.pallas_call(
        _mm_kernel,
        grid=(m // bm,),
        in_specs=[
            pl.BlockSpec((bm, k), lambda i: (i, 0)),
            pl.BlockSpec((bm, 1), lambda i: (i, 0)),
            pl.BlockSpec((k, n), lambda i: (0, 0)),
        ],
        out_specs=pl.BlockSpec((bm, n), lambda i: (i, 0)),
        out_shape=jax.ShapeDtypeStruct((m, n), out_dtype),
        compiler_params=pltpu.CompilerParams(
            dimension_semantics=("parallel",)),
    )(h, dinv, w)


def _prep_kernel(a_ref, ab_ref, s_ref):
    k = pl.program_id(1)

    @pl.when(k == 0)
    def _init():
        s_ref[...] = jnp.zeros_like(s_ref)

    blk = a_ref[...]
    ab_ref[...] = blk.astype(jnp.bfloat16)
    s_ref[...] += jnp.sum(blk, axis=1, keepdims=True)

    @pl.when(k == pl.num_programs(1) - 1)
    def _fin():
        s_ref[...] = jax.lax.rsqrt(s_ref[...] + 1.0)


def _prep(a, bm=1024):
    m = a.shape[0]
    return pl.pallas_call(
        _prep_kernel,
        grid=(m // bm, m // bm),
        in_specs=[pl.BlockSpec((bm, bm), lambda i, k: (i, k))],
        out_specs=[pl.BlockSpec((bm, bm), lambda i, k: (i, k)),
                   pl.BlockSpec((bm, 1), lambda i, k: (i, 0))],
        out_shape=[jax.ShapeDtypeStruct((m, m), jnp.bfloat16),
                   jax.ShapeDtypeStruct((m, 1), jnp.float32)],
        compiler_params=pltpu.CompilerParams(
            dimension_semantics=("parallel", "arbitrary")),
    )(a)


def _agg_kernel(a_ref, hw_ref, d_ref, b_ref, o_ref):
    i = pl.program_id(0)
    k = pl.program_id(1)

    @pl.when(k == 0)
    def _init():
        o_ref[...] = jnp.zeros_like(o_ref)

    @pl.when(k == i)
    def _self_loop():
        o_ref[...] += hw_ref[...].astype(jnp.float32)

    o_ref[...] += jnp.dot(a_ref[...], hw_ref[...],
                          preferred_element_type=jnp.float32)

    @pl.when(k == pl.num_programs(1) - 1)
    def _fin():
        o_ref[...] = jnp.maximum(d_ref[...] * o_ref[...] + b_ref[...], 0.0)


def _aggregate(a, hw, dinv, b, bm=1024):
    m = a.shape[0]
    n = hw.shape[1]
    return pl.pallas_call(
        _agg_kernel,
        grid=(m // bm, m // bm),
        in_specs=[
            pl.BlockSpec((bm, bm), lambda i, k: (i, k)),
            pl.BlockSpec((bm, n), lambda i, k: (k, 0)),
            pl.BlockSpec((bm, 1), lambda i, k: (i, 0)),
            pl.BlockSpec((1, n), lambda i, k: (0, 0)),
        ],
        out_specs=pl.BlockSpec((bm, n), lambda i, k: (i, 0)),
        out_shape=jax.ShapeDtypeStruct((m, n), jnp.float32),
        compiler_params=pltpu.CompilerParams(
            dimension_semantics=("parallel", "arbitrary")),
    )(a, hw, dinv, b)


def _proj_kernel(h_ref, w_ref, o_ref):
    o_ref[...] = jnp.dot(h_ref[...], w_ref[...],
                         preferred_element_type=jnp.float32)


def _outer_kernel(vr_ref, vt_ref, o_ref):
    o_ref[...] = jnp.maximum(vr_ref[...] * vt_ref[...], 0.0)


def _outer_relu(v, bm=400):
    vt = v.reshape(1, N)
    return pl.pallas_call(
        _outer_kernel,
        grid=(N // bm,),
        in_specs=[
            pl.BlockSpec((bm, 1), lambda i: (i, 0)),
            pl.BlockSpec((1, N), lambda i: (0, 0)),
        ],
        out_specs=pl.BlockSpec((bm, N), lambda i: (i, 0)),
        out_shape=jax.ShapeDtypeStruct((N, N), jnp.float32),
        compiler_params=pltpu.CompilerParams(
            dimension_semantics=("parallel",)),
    )(v, vt)


def kernel(x, edge_index, W1, b1, W2, b2, W3, b3, W4, b4, Wout, bout):
    src = edge_index[0].astype(jnp.int32)
    dst = edge_index[1].astype(jnp.int32)

    ones = jnp.ones(src.shape, jnp.float32)
    flat = jnp.zeros((NP * NP,), jnp.float32)
    flat = flat.at[dst * NP + src].add(ones)
    a32 = flat.reshape(NP, NP)

    a, dinv = _prep(a32)

    h = jnp.pad(x, ((0, NP - N), (0, 0)))
    for w, b in ((W1, b1), (W2, b2), (W3, b3), (W4, b4)):
        hw = _scaled_matmul(h, dinv, w)
        h = _aggregate(a, hw, dinv, b.reshape(1, -1))

    v = pl.pallas_call(
        _proj_kernel,
        in_specs=[pl.BlockSpec((NP, h.shape[1]), lambda: (0, 0)),
                  pl.BlockSpec((h.shape[1], 1), lambda: (0, 0))],
        out_specs=pl.BlockSpec((NP, 1), lambda: (0, 0)),
        out_shape=jax.ShapeDtypeStruct((NP, 1), jnp.float32),
    )(h, Wout)
    v = (v + bout)[:N]
    return _outer_relu(v)
